# baseline (speedup 1.0000x reference)
import functools

import jax
import jax.numpy as jnp
from jax import lax
from jax.experimental import pallas as pl
from jax.experimental.pallas import tpu as pltpu
from jax.experimental.pallas import tpu_sc as plsc

H = 512
W = 512
C = 3
N = H * W
NW = 32
PPW = N // NW
CH = 2048
NCHUNK = PPW // CH
SUBSZ = 256
NSUB = CH // SUBSZ
NG = CH // 64
CHBUF = 8712

_f32 = jnp.float32
_i32 = jnp.int32

_SC_PARAMS = dict(
    mesh=plsc.VectorSubcoreMesh(core_axis_name="c", subcore_axis_name="s"),
    compiler_params=pltpu.CompilerParams(needs_layout_passes=False,
                                         use_tc_tiling_on_sc=False),
)


def _sc_build_body(imp_hbm, t_hbm, ch0_v, ch1_v, ch2_v, tbA, tbB,
                   osemA, osemB):
    wid = lax.axis_index("s") * 2 + lax.axis_index("c")
    base_px = wid * PPW
    iota = lax.iota(_i32, 16)

    in_cps = [
        pltpu.async_copy(imp_hbm.at[pl.ds(base_px, CHBUF)], ch0_v, osemA),
        pltpu.async_copy(imp_hbm.at[pl.ds(N + base_px, CHBUF)], ch1_v, osemA),
    ]

    @pl.when(wid < NW - 1)
    def _():
        pltpu.sync_copy(imp_hbm.at[pl.ds(2 * N + base_px, CHBUF)], ch2_v)

    @pl.when(wid == NW - 1)
    def _():
        pltpu.sync_copy(imp_hbm.at[pl.ds(2 * N + base_px, PPW)],
                        ch2_v.at[pl.ds(0, PPW)])
        fz = jnp.full((16,), 0.0, _f32)
        for t in range((CHBUF - PPW) // 16):
            ch2_v[pl.ds(PPW + t * 16, 16)] = fz

    for cp in in_cps:
        cp.wait()

    def fill(kk, tb_v):
        def group(q, c2):
            sb = kk * CH + q * 16
            pix = q * 16 + iota
            for n, off in enumerate((0, 1, W, W + 1)):
                for c_, chv in enumerate((ch0_v, ch1_v, ch2_v)):
                    v = chv[pl.ds(sb + off, 16)]
                    plsc.store_scatter(
                        tb_v, [pix, jnp.full((16,), n * 4 + c_, _i32)], v)
            return c2

        lax.fori_loop(0, CH // 16, group, 0)

    def fire(kk, tb_v, osem):
        pltpu.async_copy(tb_v, t_hbm.at[pl.ds(base_px + kk * CH, CH)], osem)

    def drain(tb_v, osem):
        pltpu.make_async_copy(
            tb_v, t_hbm.at[pl.ds(base_px, CH)], osem).wait()

    def body(m, carry):
        k0 = 2 * m

        @pl.when(m > 0)
        def _():
            drain(tbA, osemA)

        fill(k0, tbA)
        fire(k0, tbA, osemA)

        @pl.when(m > 0)
        def _():
            drain(tbB, osemB)

        fill(k0 + 1, tbB)
        fire(k0 + 1, tbB, osemB)
        return carry

    lax.fori_loop(0, NCHUNK // 2, body, 0)
    drain(tbA, osemA)
    drain(tbB, osemB)


def _sc_undistort_body(t_hbm, par_hbm, out_hbm,
                       par_v,
                       idxA, idxB, rowsA, rowsB,
                       w00A, w01A, w10A, w11A,
                       w00B, w01B, w10B, w11B,
                       o0A, o1A, o2A, o0B, o1B, o2B,
                       gsemA, gsemB, osemA, osemB):
    wid = lax.axis_index("s") * 2 + lax.axis_index("c")
    base_px = wid * PPW
    pltpu.sync_copy(par_hbm, par_v)
    iota = lax.iota(_i32, 16)

    bufs = (
        (idxA, rowsA, (w00A, w01A, w10A, w11A), (o0A, o1A, o2A), gsemA, osemA),
        (idxB, rowsB, (w00B, w01B, w10B, w11B), (o0B, o1B, o2B), gsemB, osemB),
    )

    iotaf4 = (iota * 4).astype(_f32)
    fone = jnp.full((16,), 1.0, _f32)
    fzero = jnp.full((16,), 0.0, _f32)

    def phase1(kk, par):
        idx_v, _, wv, _, _, _ = bufs[par]
        w00_v, w01_v, w10_v, w11_v = wv
        kq = par_v[0, :]
        e1x = par_v[2, :]
        e1y = par_v[3, :]
        ybase = (base_px + kk * CH) // W

        def pgroup(q, c2):
            yrow = (ybase + (q // 8)).astype(_f32)
            yup = yrow - e1y
            yup2 = yup * yup
            for j in range(4):
                s_base = q * 64 + j * 16
                x0 = ((q % 8) * 64 + j).astype(_f32)
                xup = (x0 + iotaf4) - e1x
                q2 = xup * xup + yup2
                r = 1.0 / (1.0 - kq * q2)
                xd = xup * r + e1x
                yd = yup * r + e1y
                valid = ((xd >= 0.0) & (xd <= float(W - 1))) \
                    & ((yd >= 0.0) & (yd <= float(H - 1)))
                xt = xd.astype(_i32)
                yt = yd.astype(_i32)
                ox = xd - xt.astype(_f32)
                oy = yd - yt.astype(_f32)
                vm = jnp.where(valid, fone, fzero)
                onyv = (1.0 - oy) * vm
                oyv = oy * vm
                onx = 1.0 - ox
                xf = jnp.minimum(jnp.maximum(xt, 0), W - 1)
                yf = jnp.minimum(jnp.maximum(yt, 0), H - 1)
                idx = lax.bitwise_or(lax.shift_left(yf, 9), xf)
                idx_v[pl.ds(s_base, 16)] = idx
                w00_v[pl.ds(s_base, 16)] = onx * onyv
                w01_v[pl.ds(s_base, 16)] = ox * onyv
                w10_v[pl.ds(s_base, 16)] = onx * oyv
                w11_v[pl.ds(s_base, 16)] = ox * oyv
            return c2

        lax.fori_loop(0, NG, pgroup, 0)

    def fire_gather(par):
        idx_v, rows_v, _, _, gsem, _ = bufs[par]
        for i_ in range(NSUB):
            pltpu.async_copy(
                t_hbm.at[idx_v.at[pl.ds(i_ * SUBSZ, SUBSZ)]],
                rows_v.at[pl.ds(i_ * SUBSZ, SUBSZ)], gsem)

    def drain_gather(par):
        idx_v, rows_v, _, _, gsem, _ = bufs[par]
        pltpu.make_async_copy(t_hbm.at[idx_v], rows_v, gsem).wait()

    def combine(kk, par):
        _, rows_v, wv, ov, _, osem = bufs[par]
        w00_v, w01_v, w10_v, w11_v = wv

        def cgroup(q, c2):
            words = [jnp.full((16,), 0, _i32) for _ in range(3)]
            for j in range(4):
                s_base = q * 64 + j * 16
                pix = s_base + iota
                w00 = w00_v[pl.ds(s_base, 16)]
                w01 = w01_v[pl.ds(s_base, 16)]
                w10 = w10_v[pl.ds(s_base, 16)]
                w11 = w11_v[pl.ds(s_base, 16)]
                for c_ in range(3):
                    v00 = plsc.load_gather(
                        rows_v, [pix, jnp.full((16,), c_, _i32)])
                    v01 = plsc.load_gather(
                        rows_v, [pix, jnp.full((16,), 4 + c_, _i32)])
                    v10 = plsc.load_gather(
                        rows_v, [pix, jnp.full((16,), 8 + c_, _i32)])
                    v11 = plsc.load_gather(
                        rows_v, [pix, jnp.full((16,), 12 + c_, _i32)])
                    acc = w00 * v00 + w01 * v01 + w10 * v10 + w11 * v11
                    byte = lax.bitwise_and(acc.astype(_i32), 255)
                    words[c_] = lax.bitwise_or(
                        words[c_], lax.shift_left(byte, 8 * j))
            ov[0][pl.ds(q * 64, 64)] = plsc.bitcast(words[0], jnp.uint8)
            ov[1][pl.ds(q * 64, 64)] = plsc.bitcast(words[1], jnp.uint8)
            ov[2][pl.ds(q * 64, 64)] = plsc.bitcast(words[2], jnp.uint8)
            return c2

        lax.fori_loop(0, NG, cgroup, 0)

    def fire_out(kk, par):
        _, _, _, ov, _, osem = bufs[par]
        wbase = wid * PPW + kk * CH
        for c_ in range(3):
            pltpu.async_copy(
                ov[c_], out_hbm.at[pl.ds(c_ * N + wbase, CH)],
                osem)

    def drain_out(par):
        _, _, _, ov, _, osem = bufs[par]
        for c_ in range(3):
            pltpu.make_async_copy(
                ov[c_], out_hbm.at[pl.ds(c_ * N, CH)],
                osem).wait()

    HALF = NCHUNK // 2

    phase1(0, 0)
    fire_gather(0)

    def body(m, carry):
        k0 = 2 * m
        k1 = k0 + 1

        phase1(k1, 1)
        fire_gather(1)

        drain_gather(0)

        @pl.when(m > 0)
        def _():
            drain_out(0)

        combine(k0, 0)
        fire_out(k0, 0)

        @pl.when(m < HALF - 1)
        def _():
            phase1(k0 + 2, 0)
            fire_gather(0)

        drain_gather(1)

        @pl.when(m > 0)
        def _():
            drain_out(1)

        combine(k1, 1)
        fire_out(k1, 1)
        return carry

    lax.fori_loop(0, HALF, body, 0)
    drain_out(0)
    drain_out(1)


def _sc_build(imp):
    f = pl.kernel(
        _sc_build_body,
        out_type=jax.ShapeDtypeStruct((N, 16), _f32),
        scratch_types=[
            pltpu.VMEM((CHBUF,), _f32),
            pltpu.VMEM((CHBUF,), _f32),
            pltpu.VMEM((CHBUF,), _f32),
            pltpu.VMEM((CH, 16), _f32),
            pltpu.VMEM((CH, 16), _f32),
            pltpu.SemaphoreType.DMA,
            pltpu.SemaphoreType.DMA,
        ],
        **_SC_PARAMS,
    )
    return f(imp)


@functools.partial(jax.jit, static_argnames=())
def _undistort_sc(t, params):
    f = pl.kernel(
        _sc_undistort_body,
        out_type=jax.ShapeDtypeStruct((C * N,), jnp.uint8),
        scratch_types=(
            [pltpu.VMEM((4, 16), _f32)]
            + [pltpu.VMEM((CH,), _i32)] * 2
            + [pltpu.VMEM((CH, 16), _f32)] * 2
            + [pltpu.VMEM((CH,), _f32)] * 8
            + [pltpu.VMEM((CH,), jnp.uint8)] * 6
            + [pltpu.SemaphoreType.DMA] * 4
        ),
        **_SC_PARAMS,
    )
    return f(t, params)


def kernel(im_d, k, dx, dy):
    imp = im_d.reshape(-1)
    t = _sc_build(imp)
    params = jnp.stack([
        jnp.broadcast_to(k.reshape(1) * (1.0 / (W * W)), (16,)),
        jnp.zeros((16,), _f32),
        jnp.broadcast_to(dx.reshape(1) + (W / 2.0), (16,)),
        jnp.broadcast_to(dy.reshape(1) + (H / 2.0), (16,)),
    ]).astype(_f32)
    out = _undistort_sc(t, params)
    return out.reshape(C, H, W)


# speedup vs baseline: 1.0257x; 1.0257x over previous
import functools

import jax
import jax.numpy as jnp
from jax import lax
from jax.experimental import pallas as pl
from jax.experimental.pallas import tpu as pltpu
from jax.experimental.pallas import tpu_sc as plsc

H = 512
W = 512
C = 3
N = H * W
NW = 32
PPW = N // NW
CH = 1024
NCHUNK = PPW // CH
SUBSZ = 256
NSUB = CH // SUBSZ
NG = CH // 64
CHBUF = 8712

_f32 = jnp.float32
_i32 = jnp.int32

_SC_PARAMS = dict(
    mesh=plsc.VectorSubcoreMesh(core_axis_name="c", subcore_axis_name="s"),
    compiler_params=pltpu.CompilerParams(needs_layout_passes=False,
                                         use_tc_tiling_on_sc=False),
)


def _sc_build_body(imp_hbm, t_hbm, ch0_v, ch1_v, ch2_v, tbA, tbB,
                   osemA, osemB):
    wid = lax.axis_index("s") * 2 + lax.axis_index("c")
    base_px = wid * PPW
    iota = lax.iota(_i32, 16)

    in_cps = [
        pltpu.async_copy(imp_hbm.at[pl.ds(base_px, CHBUF)], ch0_v, osemA),
        pltpu.async_copy(imp_hbm.at[pl.ds(N + base_px, CHBUF)], ch1_v, osemA),
    ]

    @pl.when(wid < NW - 1)
    def _():
        pltpu.sync_copy(imp_hbm.at[pl.ds(2 * N + base_px, CHBUF)], ch2_v)

    @pl.when(wid == NW - 1)
    def _():
        pltpu.sync_copy(imp_hbm.at[pl.ds(2 * N + base_px, PPW)],
                        ch2_v.at[pl.ds(0, PPW)])
        fz = jnp.full((16,), 0.0, _f32)
        for t in range((CHBUF - PPW) // 16):
            ch2_v[pl.ds(PPW + t * 16, 16)] = fz

    for cp in in_cps:
        cp.wait()

    def fill(kk, tb_v):
        def group(q, c2):
            sb = kk * CH + q * 16
            pix = q * 16 + iota
            for n, off in enumerate((0, 1, W, W + 1)):
                for c_, chv in enumerate((ch0_v, ch1_v, ch2_v)):
                    v = chv[pl.ds(sb + off, 16)]
                    plsc.store_scatter(
                        tb_v, [pix, jnp.full((16,), n * 4 + c_, _i32)], v)
            return c2

        lax.fori_loop(0, CH // 16, group, 0)

    def fire(kk, tb_v, osem):
        pltpu.async_copy(tb_v, t_hbm.at[pl.ds(base_px + kk * CH, CH)], osem)

    def drain(tb_v, osem):
        pltpu.make_async_copy(
            tb_v, t_hbm.at[pl.ds(base_px, CH)], osem).wait()

    def body(m, carry):
        k0 = 2 * m

        @pl.when(m > 0)
        def _():
            drain(tbA, osemA)

        fill(k0, tbA)
        fire(k0, tbA, osemA)

        @pl.when(m > 0)
        def _():
            drain(tbB, osemB)

        fill(k0 + 1, tbB)
        fire(k0 + 1, tbB, osemB)
        return carry

    lax.fori_loop(0, NCHUNK // 2, body, 0)
    drain(tbA, osemA)
    drain(tbB, osemB)


def _sc_undistort_body(t_hbm, par_hbm, out_hbm,
                       par_v,
                       idxA, idxB, rowsA, rowsB,
                       w00A, w01A, w10A, w11A,
                       w00B, w01B, w10B, w11B,
                       o0A, o1A, o2A, o0B, o1B, o2B,
                       gsemA, gsemB, osemA, osemB):
    wid = lax.axis_index("s") * 2 + lax.axis_index("c")
    base_px = wid * PPW
    pltpu.sync_copy(par_hbm, par_v)
    iota = lax.iota(_i32, 16)

    bufs = (
        (idxA, rowsA, (w00A, w01A, w10A, w11A), (o0A, o1A, o2A), gsemA, osemA),
        (idxB, rowsB, (w00B, w01B, w10B, w11B), (o0B, o1B, o2B), gsemB, osemB),
    )

    iotaf4 = (iota * 4).astype(_f32)
    fone = jnp.full((16,), 1.0, _f32)
    fzero = jnp.full((16,), 0.0, _f32)

    def phase1(kk, par):
        idx_v, _, wv, _, _, _ = bufs[par]
        w00_v, w01_v, w10_v, w11_v = wv
        kq = par_v[0, :]
        e1x = par_v[2, :]
        e1y = par_v[3, :]
        ybase = (base_px + kk * CH) // W

        def pgroup(q, c2):
            yrow = (ybase + (q // 8)).astype(_f32)
            yup = yrow - e1y
            yup2 = yup * yup
            for j in range(4):
                s_base = q * 64 + j * 16
                x0 = ((q % 8) * 64 + j).astype(_f32)
                xup = (x0 + iotaf4) - e1x
                q2 = xup * xup + yup2
                r = 1.0 / (1.0 - kq * q2)
                xd = xup * r + e1x
                yd = yup * r + e1y
                valid = ((xd >= 0.0) & (xd <= float(W - 1))) \
                    & ((yd >= 0.0) & (yd <= float(H - 1)))
                xt = xd.astype(_i32)
                yt = yd.astype(_i32)
                ox = xd - xt.astype(_f32)
                oy = yd - yt.astype(_f32)
                vm = jnp.where(valid, fone, fzero)
                onyv = (1.0 - oy) * vm
                oyv = oy * vm
                onx = 1.0 - ox
                xf = jnp.minimum(jnp.maximum(xt, 0), W - 1)
                yf = jnp.minimum(jnp.maximum(yt, 0), H - 1)
                idx = lax.bitwise_or(lax.shift_left(yf, 9), xf)
                idx_v[pl.ds(s_base, 16)] = idx
                w00_v[pl.ds(s_base, 16)] = onx * onyv
                w01_v[pl.ds(s_base, 16)] = ox * onyv
                w10_v[pl.ds(s_base, 16)] = onx * oyv
                w11_v[pl.ds(s_base, 16)] = ox * oyv
            return c2

        lax.fori_loop(0, NG, pgroup, 0)

    def fire_gather(par):
        idx_v, rows_v, _, _, gsem, _ = bufs[par]
        for i_ in range(NSUB):
            pltpu.async_copy(
                t_hbm.at[idx_v.at[pl.ds(i_ * SUBSZ, SUBSZ)]],
                rows_v.at[pl.ds(i_ * SUBSZ, SUBSZ)], gsem)

    def drain_gather(par):
        idx_v, rows_v, _, _, gsem, _ = bufs[par]
        pltpu.make_async_copy(t_hbm.at[idx_v], rows_v, gsem).wait()

    def combine(kk, par):
        _, rows_v, wv, ov, _, osem = bufs[par]
        w00_v, w01_v, w10_v, w11_v = wv

        def cgroup(q, c2):
            words = [jnp.full((16,), 0, _i32) for _ in range(3)]
            for j in range(4):
                s_base = q * 64 + j * 16
                pix = s_base + iota
                w00 = w00_v[pl.ds(s_base, 16)]
                w01 = w01_v[pl.ds(s_base, 16)]
                w10 = w10_v[pl.ds(s_base, 16)]
                w11 = w11_v[pl.ds(s_base, 16)]
                for c_ in range(3):
                    v00 = plsc.load_gather(
                        rows_v, [pix, jnp.full((16,), c_, _i32)])
                    v01 = plsc.load_gather(
                        rows_v, [pix, jnp.full((16,), 4 + c_, _i32)])
                    v10 = plsc.load_gather(
                        rows_v, [pix, jnp.full((16,), 8 + c_, _i32)])
                    v11 = plsc.load_gather(
                        rows_v, [pix, jnp.full((16,), 12 + c_, _i32)])
                    acc = w00 * v00 + w01 * v01 + w10 * v10 + w11 * v11
                    byte = lax.bitwise_and(acc.astype(_i32), 255)
                    words[c_] = lax.bitwise_or(
                        words[c_], lax.shift_left(byte, 8 * j))
            ov[0][pl.ds(q * 64, 64)] = plsc.bitcast(words[0], jnp.uint8)
            ov[1][pl.ds(q * 64, 64)] = plsc.bitcast(words[1], jnp.uint8)
            ov[2][pl.ds(q * 64, 64)] = plsc.bitcast(words[2], jnp.uint8)
            return c2

        lax.fori_loop(0, NG, cgroup, 0)

    def fire_out(kk, par):
        _, _, _, ov, _, osem = bufs[par]
        wbase = wid * PPW + kk * CH
        for c_ in range(3):
            pltpu.async_copy(
                ov[c_], out_hbm.at[pl.ds(c_ * N + wbase, CH)],
                osem)

    def drain_out(par):
        _, _, _, ov, _, osem = bufs[par]
        for c_ in range(3):
            pltpu.make_async_copy(
                ov[c_], out_hbm.at[pl.ds(c_ * N, CH)],
                osem).wait()

    HALF = NCHUNK // 2

    phase1(0, 0)
    fire_gather(0)

    def body(m, carry):
        k0 = 2 * m
        k1 = k0 + 1

        phase1(k1, 1)
        fire_gather(1)

        drain_gather(0)

        @pl.when(m > 0)
        def _():
            drain_out(0)

        combine(k0, 0)
        fire_out(k0, 0)

        @pl.when(m < HALF - 1)
        def _():
            phase1(k0 + 2, 0)
            fire_gather(0)

        drain_gather(1)

        @pl.when(m > 0)
        def _():
            drain_out(1)

        combine(k1, 1)
        fire_out(k1, 1)
        return carry

    lax.fori_loop(0, HALF, body, 0)
    drain_out(0)
    drain_out(1)


def _sc_build(imp):
    f = pl.kernel(
        _sc_build_body,
        out_type=jax.ShapeDtypeStruct((N, 16), _f32),
        scratch_types=[
            pltpu.VMEM((CHBUF,), _f32),
            pltpu.VMEM((CHBUF,), _f32),
            pltpu.VMEM((CHBUF,), _f32),
            pltpu.VMEM((CH, 16), _f32),
            pltpu.VMEM((CH, 16), _f32),
            pltpu.SemaphoreType.DMA,
            pltpu.SemaphoreType.DMA,
        ],
        **_SC_PARAMS,
    )
    return f(imp)


@functools.partial(jax.jit, static_argnames=())
def _undistort_sc(t, params):
    f = pl.kernel(
        _sc_undistort_body,
        out_type=jax.ShapeDtypeStruct((C * N,), jnp.uint8),
        scratch_types=(
            [pltpu.VMEM((4, 16), _f32)]
            + [pltpu.VMEM((CH,), _i32)] * 2
            + [pltpu.VMEM((CH, 16), _f32)] * 2
            + [pltpu.VMEM((CH,), _f32)] * 8
            + [pltpu.VMEM((CH,), jnp.uint8)] * 6
            + [pltpu.SemaphoreType.DMA] * 4
        ),
        **_SC_PARAMS,
    )
    return f(t, params)


def kernel(im_d, k, dx, dy):
    imp = im_d.reshape(-1)
    t = _sc_build(imp)
    params = jnp.stack([
        jnp.broadcast_to(k.reshape(1) * (1.0 / (W * W)), (16,)),
        jnp.zeros((16,), _f32),
        jnp.broadcast_to(dx.reshape(1) + (W / 2.0), (16,)),
        jnp.broadcast_to(dy.reshape(1) + (H / 2.0), (16,)),
    ]).astype(_f32)
    out = _undistort_sc(t, params)
    return out.reshape(C, H, W)


# speedup vs baseline: 1.1330x; 1.1046x over previous
import functools

import jax
import jax.numpy as jnp
from jax import lax
from jax.experimental import pallas as pl
from jax.experimental.pallas import tpu as pltpu
from jax.experimental.pallas import tpu_sc as plsc

H = 512
W = 512
C = 3
N = H * W
NW = 32
PPW = N // NW
CH = 1024
NCHUNK = PPW // CH
SUBSZ = 256
NSUB = CH // SUBSZ
NG = CH // 64
CHBUF = 8712

_f32 = jnp.float32
_i32 = jnp.int32

_SC_PARAMS = dict(
    mesh=plsc.VectorSubcoreMesh(core_axis_name="c", subcore_axis_name="s"),
    compiler_params=pltpu.CompilerParams(needs_layout_passes=False,
                                         use_tc_tiling_on_sc=False),
)


def _sc_build_body(imp_hbm, t_hbm, ch0_v, ch1_v, ch2_v, tbA, tbB,
                   osemA, osemB):
    wid = lax.axis_index("s") * 2 + lax.axis_index("c")
    base_px = wid * PPW
    iota = lax.iota(_i32, 16)

    in_cps = [
        pltpu.async_copy(imp_hbm.at[pl.ds(base_px, CHBUF)], ch0_v, osemA),
        pltpu.async_copy(imp_hbm.at[pl.ds(N + base_px, CHBUF)], ch1_v, osemA),
    ]

    @pl.when(wid < NW - 1)
    def _():
        pltpu.sync_copy(imp_hbm.at[pl.ds(2 * N + base_px, CHBUF)], ch2_v)

    @pl.when(wid == NW - 1)
    def _():
        pltpu.sync_copy(imp_hbm.at[pl.ds(2 * N + base_px, PPW)],
                        ch2_v.at[pl.ds(0, PPW)])
        fz = jnp.full((16,), 0.0, _f32)
        for t in range((CHBUF - PPW) // 16):
            ch2_v[pl.ds(PPW + t * 16, 16)] = fz

    for cp in in_cps:
        cp.wait()

    def fill(kk, tb_v):
        def group(q, c2):
            sb = kk * CH + q * 16
            pix = q * 16 + iota
            for n, off in enumerate((0, 1)):
                for c_, chv in enumerate((ch0_v, ch1_v, ch2_v)):
                    v = chv[pl.ds(sb + off, 16)]
                    plsc.store_scatter(
                        tb_v, [pix, jnp.full((16,), n * 4 + c_, _i32)], v)
            return c2

        lax.fori_loop(0, CH // 16, group, 0)

    def fire(kk, tb_v, osem):
        pltpu.async_copy(tb_v, t_hbm.at[pl.ds(base_px + kk * CH, CH)], osem)

    def drain(tb_v, osem):
        pltpu.make_async_copy(
            tb_v, t_hbm.at[pl.ds(base_px, CH)], osem).wait()

    def body(m, carry):
        k0 = 2 * m

        @pl.when(m > 0)
        def _():
            drain(tbA, osemA)

        fill(k0, tbA)
        fire(k0, tbA, osemA)

        @pl.when(m > 0)
        def _():
            drain(tbB, osemB)

        fill(k0 + 1, tbB)
        fire(k0 + 1, tbB, osemB)
        return carry

    lax.fori_loop(0, NCHUNK // 2, body, 0)
    drain(tbA, osemA)
    drain(tbB, osemB)

    @pl.when(wid == NW - 1)
    def _():
        fz = jnp.full((16,), 0.0, _f32)

        def zgroup(rg, c2):
            pix = rg * 16 + iota
            for col in range(8):
                plsc.store_scatter(
                    tbA, [pix, jnp.full((16,), col, _i32)], fz)
            return c2

        lax.fori_loop(0, 512 // 16, zgroup, 0)
        pltpu.sync_copy(tbA.at[pl.ds(0, 512)], t_hbm.at[pl.ds(N, 512)])


def _sc_undistort_body(t_hbm, par_hbm, out_hbm,
                       par_v,
                       idxA, idxB, rowsTA, rowsBA, rowsTB, rowsBB,
                       w00A, w01A, w10A, w11A,
                       w00B, w01B, w10B, w11B,
                       o0A, o1A, o2A, o0B, o1B, o2B,
                       gsemA, gsemB, osemA, osemB):
    wid = lax.axis_index("s") * 2 + lax.axis_index("c")
    base_px = wid * PPW
    pltpu.sync_copy(par_hbm, par_v)
    iota = lax.iota(_i32, 16)

    bufs = (
        (idxA, (rowsTA, rowsBA), (w00A, w01A, w10A, w11A),
         (o0A, o1A, o2A), gsemA, osemA),
        (idxB, (rowsTB, rowsBB), (w00B, w01B, w10B, w11B),
         (o0B, o1B, o2B), gsemB, osemB),
    )

    iotaf4 = (iota * 4).astype(_f32)
    fone = jnp.full((16,), 1.0, _f32)
    fzero = jnp.full((16,), 0.0, _f32)

    def phase1(kk, par):
        idx_v, _, wv, _, _, _ = bufs[par]
        w00_v, w01_v, w10_v, w11_v = wv
        kq = par_v[0, :]
        e1x = par_v[2, :]
        e1y = par_v[3, :]
        ybase = (base_px + kk * CH) // W

        def pgroup(q, c2):
            yrow = (ybase + (q // 8)).astype(_f32)
            yup = yrow - e1y
            yup2 = yup * yup
            for j in range(4):
                s_base = q * 64 + j * 16
                x0 = ((q % 8) * 64 + j).astype(_f32)
                xup = (x0 + iotaf4) - e1x
                q2 = xup * xup + yup2
                r = 1.0 / (1.0 - kq * q2)
                xd = xup * r + e1x
                yd = yup * r + e1y
                valid = ((xd >= 0.0) & (xd <= float(W - 1))) \
                    & ((yd >= 0.0) & (yd <= float(H - 1)))
                xt = xd.astype(_i32)
                yt = yd.astype(_i32)
                ox = xd - xt.astype(_f32)
                oy = yd - yt.astype(_f32)
                vm = jnp.where(valid, fone, fzero)
                onyv = (1.0 - oy) * vm
                oyv = oy * vm
                onx = 1.0 - ox
                xf = jnp.minimum(jnp.maximum(xt, 0), W - 1)
                yf = jnp.minimum(jnp.maximum(yt, 0), H - 1)
                idx = lax.bitwise_or(lax.shift_left(yf, 9), xf)
                idx_v[pl.ds(s_base, 16)] = idx
                idx_v[pl.ds(CH + s_base, 16)] = idx + W
                w00_v[pl.ds(s_base, 16)] = onx * onyv
                w01_v[pl.ds(s_base, 16)] = ox * onyv
                w10_v[pl.ds(s_base, 16)] = onx * oyv
                w11_v[pl.ds(s_base, 16)] = ox * oyv
            return c2

        lax.fori_loop(0, NG, pgroup, 0)

    def fire_gather(par):
        idx_v, (rowsT, rowsB), _, _, gsem, _ = bufs[par]
        for half, rows_v in enumerate((rowsT, rowsB)):
            for i_ in range(NSUB):
                pltpu.async_copy(
                    t_hbm.at[idx_v.at[pl.ds(half * CH + i_ * SUBSZ, SUBSZ)]],
                    rows_v.at[pl.ds(i_ * SUBSZ, SUBSZ)], gsem)

    def drain_gather(par):
        idx_v, (rowsT, rowsB), _, _, gsem, _ = bufs[par]
        pltpu.make_async_copy(t_hbm.at[idx_v.at[pl.ds(0, CH)]],
                              rowsT, gsem).wait()
        pltpu.make_async_copy(t_hbm.at[idx_v.at[pl.ds(CH, CH)]],
                              rowsB, gsem).wait()

    def combine(kk, par):
        _, (rowsT, rowsB), wv, ov, _, osem = bufs[par]
        w00_v, w01_v, w10_v, w11_v = wv

        def cgroup(q, c2):
            words = [jnp.full((16,), 0, _i32) for _ in range(3)]
            for j in range(4):
                s_base = q * 64 + j * 16
                pix = s_base + iota
                w00 = w00_v[pl.ds(s_base, 16)]
                w01 = w01_v[pl.ds(s_base, 16)]
                w10 = w10_v[pl.ds(s_base, 16)]
                w11 = w11_v[pl.ds(s_base, 16)]
                for c_ in range(3):
                    v00 = plsc.load_gather(
                        rowsT, [pix, jnp.full((16,), c_, _i32)])
                    v01 = plsc.load_gather(
                        rowsT, [pix, jnp.full((16,), 4 + c_, _i32)])
                    v10 = plsc.load_gather(
                        rowsB, [pix, jnp.full((16,), c_, _i32)])
                    v11 = plsc.load_gather(
                        rowsB, [pix, jnp.full((16,), 4 + c_, _i32)])
                    acc = w00 * v00 + w01 * v01 + w10 * v10 + w11 * v11
                    byte = lax.bitwise_and(acc.astype(_i32), 255)
                    words[c_] = lax.bitwise_or(
                        words[c_], lax.shift_left(byte, 8 * j))
            ov[0][pl.ds(q * 64, 64)] = plsc.bitcast(words[0], jnp.uint8)
            ov[1][pl.ds(q * 64, 64)] = plsc.bitcast(words[1], jnp.uint8)
            ov[2][pl.ds(q * 64, 64)] = plsc.bitcast(words[2], jnp.uint8)
            return c2

        lax.fori_loop(0, NG, cgroup, 0)

    def fire_out(kk, par):
        _, _, _, ov, _, osem = bufs[par]
        wbase = wid * PPW + kk * CH
        for c_ in range(3):
            pltpu.async_copy(
                ov[c_], out_hbm.at[pl.ds(c_ * N + wbase, CH)],
                osem)

    def drain_out(par):
        _, _, _, ov, _, osem = bufs[par]
        for c_ in range(3):
            pltpu.make_async_copy(
                ov[c_], out_hbm.at[pl.ds(c_ * N, CH)],
                osem).wait()

    HALF = NCHUNK // 2

    phase1(0, 0)
    fire_gather(0)

    def body(m, carry):
        k0 = 2 * m
        k1 = k0 + 1

        phase1(k1, 1)
        fire_gather(1)

        drain_gather(0)

        @pl.when(m > 0)
        def _():
            drain_out(0)

        combine(k0, 0)
        fire_out(k0, 0)

        @pl.when(m < HALF - 1)
        def _():
            phase1(k0 + 2, 0)
            fire_gather(0)

        drain_gather(1)

        @pl.when(m > 0)
        def _():
            drain_out(1)

        combine(k1, 1)
        fire_out(k1, 1)
        return carry

    lax.fori_loop(0, HALF, body, 0)
    drain_out(0)
    drain_out(1)


def _sc_build(imp):
    f = pl.kernel(
        _sc_build_body,
        out_type=jax.ShapeDtypeStruct((N + W, 8), _f32),
        scratch_types=[
            pltpu.VMEM((CHBUF,), _f32),
            pltpu.VMEM((CHBUF,), _f32),
            pltpu.VMEM((CHBUF,), _f32),
            pltpu.VMEM((CH, 8), _f32),
            pltpu.VMEM((CH, 8), _f32),
            pltpu.SemaphoreType.DMA,
            pltpu.SemaphoreType.DMA,
        ],
        **_SC_PARAMS,
    )
    return f(imp)


@functools.partial(jax.jit, static_argnames=())
def _undistort_sc(t, params):
    f = pl.kernel(
        _sc_undistort_body,
        out_type=jax.ShapeDtypeStruct((C * N,), jnp.uint8),
        scratch_types=(
            [pltpu.VMEM((4, 16), _f32)]
            + [pltpu.VMEM((2 * CH,), _i32)] * 2
            + [pltpu.VMEM((CH, 8), _f32)] * 4
            + [pltpu.VMEM((CH,), _f32)] * 8
            + [pltpu.VMEM((CH,), jnp.uint8)] * 6
            + [pltpu.SemaphoreType.DMA] * 4
        ),
        **_SC_PARAMS,
    )
    return f(t, params)


def kernel(im_d, k, dx, dy):
    imp = im_d.reshape(-1)
    t = _sc_build(imp)
    params = jnp.stack([
        jnp.broadcast_to(k.reshape(1) * (1.0 / (W * W)), (16,)),
        jnp.zeros((16,), _f32),
        jnp.broadcast_to(dx.reshape(1) + (W / 2.0), (16,)),
        jnp.broadcast_to(dy.reshape(1) + (H / 2.0), (16,)),
    ]).astype(_f32)
    out = _undistort_sc(t, params)
    return out.reshape(C, H, W)


# speedup vs baseline: 1.1332x; 1.0002x over previous
import functools

import jax
import jax.numpy as jnp
from jax import lax
from jax.experimental import pallas as pl
from jax.experimental.pallas import tpu as pltpu
from jax.experimental.pallas import tpu_sc as plsc

H = 512
W = 512
C = 3
N = H * W
NW = 32
PPW = N // NW
CH = 1024
NCHUNK = PPW // CH
SUBSZ = 1024
NSUB = CH // SUBSZ
NG = CH // 64
CHBUF = 8712

_f32 = jnp.float32
_i32 = jnp.int32

_SC_PARAMS = dict(
    mesh=plsc.VectorSubcoreMesh(core_axis_name="c", subcore_axis_name="s"),
    compiler_params=pltpu.CompilerParams(needs_layout_passes=False,
                                         use_tc_tiling_on_sc=False),
)


def _sc_build_body(imp_hbm, t_hbm, ch0_v, ch1_v, ch2_v, tbA, tbB,
                   osemA, osemB):
    wid = lax.axis_index("s") * 2 + lax.axis_index("c")
    base_px = wid * PPW
    iota = lax.iota(_i32, 16)

    in_cps = [
        pltpu.async_copy(imp_hbm.at[pl.ds(base_px, CHBUF)], ch0_v, osemA),
        pltpu.async_copy(imp_hbm.at[pl.ds(N + base_px, CHBUF)], ch1_v, osemA),
    ]

    @pl.when(wid < NW - 1)
    def _():
        pltpu.sync_copy(imp_hbm.at[pl.ds(2 * N + base_px, CHBUF)], ch2_v)

    @pl.when(wid == NW - 1)
    def _():
        pltpu.sync_copy(imp_hbm.at[pl.ds(2 * N + base_px, PPW)],
                        ch2_v.at[pl.ds(0, PPW)])
        fz = jnp.full((16,), 0.0, _f32)
        for t in range((CHBUF - PPW) // 16):
            ch2_v[pl.ds(PPW + t * 16, 16)] = fz

    for cp in in_cps:
        cp.wait()

    def fill(kk, tb_v):
        def group(q, c2):
            sb = kk * CH + q * 16
            pix = q * 16 + iota
            for n, off in enumerate((0, 1)):
                for c_, chv in enumerate((ch0_v, ch1_v, ch2_v)):
                    v = chv[pl.ds(sb + off, 16)]
                    plsc.store_scatter(
                        tb_v, [pix, jnp.full((16,), n * 4 + c_, _i32)], v)
            return c2

        lax.fori_loop(0, CH // 16, group, 0)

    def fire(kk, tb_v, osem):
        pltpu.async_copy(tb_v, t_hbm.at[pl.ds(base_px + kk * CH, CH)], osem)

    def drain(tb_v, osem):
        pltpu.make_async_copy(
            tb_v, t_hbm.at[pl.ds(base_px, CH)], osem).wait()

    def body(m, carry):
        k0 = 2 * m

        @pl.when(m > 0)
        def _():
            drain(tbA, osemA)

        fill(k0, tbA)
        fire(k0, tbA, osemA)

        @pl.when(m > 0)
        def _():
            drain(tbB, osemB)

        fill(k0 + 1, tbB)
        fire(k0 + 1, tbB, osemB)
        return carry

    lax.fori_loop(0, NCHUNK // 2, body, 0)
    drain(tbA, osemA)
    drain(tbB, osemB)

    @pl.when(wid == NW - 1)
    def _():
        fz = jnp.full((16,), 0.0, _f32)

        def zgroup(rg, c2):
            pix = rg * 16 + iota
            for col in range(8):
                plsc.store_scatter(
                    tbA, [pix, jnp.full((16,), col, _i32)], fz)
            return c2

        lax.fori_loop(0, 512 // 16, zgroup, 0)
        pltpu.sync_copy(tbA.at[pl.ds(0, 512)], t_hbm.at[pl.ds(N, 512)])


def _sc_undistort_body(t_hbm, par_hbm, out_hbm,
                       par_v,
                       idxA, idxB, rowsTA, rowsBA, rowsTB, rowsBB,
                       w00A, w01A, w10A, w11A,
                       w00B, w01B, w10B, w11B,
                       o0A, o1A, o2A, o0B, o1B, o2B,
                       gsemA, gsemB, osemA, osemB):
    wid = lax.axis_index("s") * 2 + lax.axis_index("c")
    base_px = wid * PPW
    pltpu.sync_copy(par_hbm, par_v)
    iota = lax.iota(_i32, 16)

    bufs = (
        (idxA, (rowsTA, rowsBA), (w00A, w01A, w10A, w11A),
         (o0A, o1A, o2A), gsemA, osemA),
        (idxB, (rowsTB, rowsBB), (w00B, w01B, w10B, w11B),
         (o0B, o1B, o2B), gsemB, osemB),
    )

    iotaf4 = (iota * 4).astype(_f32)
    fone = jnp.full((16,), 1.0, _f32)
    fzero = jnp.full((16,), 0.0, _f32)

    def phase1(kk, par):
        idx_v, _, wv, _, _, _ = bufs[par]
        w00_v, w01_v, w10_v, w11_v = wv
        kq = par_v[0, :]
        e1x = par_v[2, :]
        e1y = par_v[3, :]
        ybase = (base_px + kk * CH) // W

        def pgroup(q, c2):
            yrow = (ybase + (q // 8)).astype(_f32)
            yup = yrow - e1y
            yup2 = yup * yup
            for j in range(4):
                s_base = q * 64 + j * 16
                x0 = ((q % 8) * 64 + j).astype(_f32)
                xup = (x0 + iotaf4) - e1x
                q2 = xup * xup + yup2
                r = 1.0 / (1.0 - kq * q2)
                xd = xup * r + e1x
                yd = yup * r + e1y
                valid = ((xd >= 0.0) & (xd <= float(W - 1))) \
                    & ((yd >= 0.0) & (yd <= float(H - 1)))
                xt = xd.astype(_i32)
                yt = yd.astype(_i32)
                ox = xd - xt.astype(_f32)
                oy = yd - yt.astype(_f32)
                vm = jnp.where(valid, fone, fzero)
                onyv = (1.0 - oy) * vm
                oyv = oy * vm
                onx = 1.0 - ox
                xf = jnp.minimum(jnp.maximum(xt, 0), W - 1)
                yf = jnp.minimum(jnp.maximum(yt, 0), H - 1)
                idx = lax.bitwise_or(lax.shift_left(yf, 9), xf)
                idx_v[pl.ds(s_base, 16)] = idx
                idx_v[pl.ds(CH + s_base, 16)] = idx + W
                w00_v[pl.ds(s_base, 16)] = onx * onyv
                w01_v[pl.ds(s_base, 16)] = ox * onyv
                w10_v[pl.ds(s_base, 16)] = onx * oyv
                w11_v[pl.ds(s_base, 16)] = ox * oyv
            return c2

        lax.fori_loop(0, NG, pgroup, 0)

    def fire_gather(par):
        idx_v, (rowsT, rowsB), _, _, gsem, _ = bufs[par]
        for half, rows_v in enumerate((rowsT, rowsB)):
            for i_ in range(NSUB):
                pltpu.async_copy(
                    t_hbm.at[idx_v.at[pl.ds(half * CH + i_ * SUBSZ, SUBSZ)]],
                    rows_v.at[pl.ds(i_ * SUBSZ, SUBSZ)], gsem)

    def drain_gather(par):
        idx_v, (rowsT, rowsB), _, _, gsem, _ = bufs[par]
        pltpu.make_async_copy(t_hbm.at[idx_v.at[pl.ds(0, CH)]],
                              rowsT, gsem).wait()
        pltpu.make_async_copy(t_hbm.at[idx_v.at[pl.ds(CH, CH)]],
                              rowsB, gsem).wait()

    def combine(kk, par):
        _, (rowsT, rowsB), wv, ov, _, osem = bufs[par]
        w00_v, w01_v, w10_v, w11_v = wv

        def cgroup(q, c2):
            words = [jnp.full((16,), 0, _i32) for _ in range(3)]
            for j in range(4):
                s_base = q * 64 + j * 16
                pix = s_base + iota
                w00 = w00_v[pl.ds(s_base, 16)]
                w01 = w01_v[pl.ds(s_base, 16)]
                w10 = w10_v[pl.ds(s_base, 16)]
                w11 = w11_v[pl.ds(s_base, 16)]
                for c_ in range(3):
                    v00 = plsc.load_gather(
                        rowsT, [pix, jnp.full((16,), c_, _i32)])
                    v01 = plsc.load_gather(
                        rowsT, [pix, jnp.full((16,), 4 + c_, _i32)])
                    v10 = plsc.load_gather(
                        rowsB, [pix, jnp.full((16,), c_, _i32)])
                    v11 = plsc.load_gather(
                        rowsB, [pix, jnp.full((16,), 4 + c_, _i32)])
                    acc = w00 * v00 + w01 * v01 + w10 * v10 + w11 * v11
                    byte = lax.bitwise_and(acc.astype(_i32), 255)
                    words[c_] = lax.bitwise_or(
                        words[c_], lax.shift_left(byte, 8 * j))
            ov[0][pl.ds(q * 64, 64)] = plsc.bitcast(words[0], jnp.uint8)
            ov[1][pl.ds(q * 64, 64)] = plsc.bitcast(words[1], jnp.uint8)
            ov[2][pl.ds(q * 64, 64)] = plsc.bitcast(words[2], jnp.uint8)
            return c2

        lax.fori_loop(0, NG, cgroup, 0)

    def fire_out(kk, par):
        _, _, _, ov, _, osem = bufs[par]
        wbase = wid * PPW + kk * CH
        for c_ in range(3):
            pltpu.async_copy(
                ov[c_], out_hbm.at[pl.ds(c_ * N + wbase, CH)],
                osem)

    def drain_out(par):
        _, _, _, ov, _, osem = bufs[par]
        for c_ in range(3):
            pltpu.make_async_copy(
                ov[c_], out_hbm.at[pl.ds(c_ * N, CH)],
                osem).wait()

    HALF = NCHUNK // 2

    phase1(0, 0)
    fire_gather(0)

    def body(m, carry):
        k0 = 2 * m
        k1 = k0 + 1

        phase1(k1, 1)
        fire_gather(1)

        drain_gather(0)

        @pl.when(m > 0)
        def _():
            drain_out(0)

        combine(k0, 0)
        fire_out(k0, 0)

        @pl.when(m < HALF - 1)
        def _():
            phase1(k0 + 2, 0)
            fire_gather(0)

        drain_gather(1)

        @pl.when(m > 0)
        def _():
            drain_out(1)

        combine(k1, 1)
        fire_out(k1, 1)
        return carry

    lax.fori_loop(0, HALF, body, 0)
    drain_out(0)
    drain_out(1)


def _sc_build(imp):
    f = pl.kernel(
        _sc_build_body,
        out_type=jax.ShapeDtypeStruct((N + W, 8), _f32),
        scratch_types=[
            pltpu.VMEM((CHBUF,), _f32),
            pltpu.VMEM((CHBUF,), _f32),
            pltpu.VMEM((CHBUF,), _f32),
            pltpu.VMEM((CH, 8), _f32),
            pltpu.VMEM((CH, 8), _f32),
            pltpu.SemaphoreType.DMA,
            pltpu.SemaphoreType.DMA,
        ],
        **_SC_PARAMS,
    )
    return f(imp)


@functools.partial(jax.jit, static_argnames=())
def _undistort_sc(t, params):
    f = pl.kernel(
        _sc_undistort_body,
        out_type=jax.ShapeDtypeStruct((C * N,), jnp.uint8),
        scratch_types=(
            [pltpu.VMEM((4, 16), _f32)]
            + [pltpu.VMEM((2 * CH,), _i32)] * 2
            + [pltpu.VMEM((CH, 8), _f32)] * 4
            + [pltpu.VMEM((CH,), _f32)] * 8
            + [pltpu.VMEM((CH,), jnp.uint8)] * 6
            + [pltpu.SemaphoreType.DMA] * 4
        ),
        **_SC_PARAMS,
    )
    return f(t, params)


def kernel(im_d, k, dx, dy):
    imp = im_d.reshape(-1)
    t = _sc_build(imp)
    params = jnp.stack([
        jnp.broadcast_to(k.reshape(1) * (1.0 / (W * W)), (16,)),
        jnp.zeros((16,), _f32),
        jnp.broadcast_to(dx.reshape(1) + (W / 2.0), (16,)),
        jnp.broadcast_to(dy.reshape(1) + (H / 2.0), (16,)),
    ]).astype(_f32)
    out = _undistort_sc(t, params)
    return out.reshape(C, H, W)


# speedup vs baseline: 1.1490x; 1.0139x over previous
import functools

import jax
import jax.numpy as jnp
from jax import lax
from jax.experimental import pallas as pl
from jax.experimental.pallas import tpu as pltpu
from jax.experimental.pallas import tpu_sc as plsc

H = 512
W = 512
C = 3
N = H * W
NW = 32
PPW = N // NW
CH = 512
NCHUNK = PPW // CH
SUBSZ = 512
NSUB = CH // SUBSZ
NG = CH // 64
CHBUF = 8712

_f32 = jnp.float32
_i32 = jnp.int32

_SC_PARAMS = dict(
    mesh=plsc.VectorSubcoreMesh(core_axis_name="c", subcore_axis_name="s"),
    compiler_params=pltpu.CompilerParams(needs_layout_passes=False,
                                         use_tc_tiling_on_sc=False),
)


def _sc_build_body(imp_hbm, t_hbm, ch0_v, ch1_v, ch2_v, tbA, tbB,
                   osemA, osemB):
    wid = lax.axis_index("s") * 2 + lax.axis_index("c")
    base_px = wid * PPW
    iota = lax.iota(_i32, 16)

    in_cps = [
        pltpu.async_copy(imp_hbm.at[pl.ds(base_px, CHBUF)], ch0_v, osemA),
        pltpu.async_copy(imp_hbm.at[pl.ds(N + base_px, CHBUF)], ch1_v, osemA),
    ]

    @pl.when(wid < NW - 1)
    def _():
        pltpu.sync_copy(imp_hbm.at[pl.ds(2 * N + base_px, CHBUF)], ch2_v)

    @pl.when(wid == NW - 1)
    def _():
        pltpu.sync_copy(imp_hbm.at[pl.ds(2 * N + base_px, PPW)],
                        ch2_v.at[pl.ds(0, PPW)])
        fz = jnp.full((16,), 0.0, _f32)
        for t in range((CHBUF - PPW) // 16):
            ch2_v[pl.ds(PPW + t * 16, 16)] = fz

    for cp in in_cps:
        cp.wait()

    def fill(kk, tb_v):
        def group(q, c2):
            sb = kk * CH + q * 16
            pix = q * 16 + iota
            for n, off in enumerate((0, 1)):
                for c_, chv in enumerate((ch0_v, ch1_v, ch2_v)):
                    v = chv[pl.ds(sb + off, 16)]
                    plsc.store_scatter(
                        tb_v, [pix, jnp.full((16,), n * 4 + c_, _i32)], v)
            return c2

        lax.fori_loop(0, CH // 16, group, 0)

    def fire(kk, tb_v, osem):
        pltpu.async_copy(tb_v, t_hbm.at[pl.ds(base_px + kk * CH, CH)], osem)

    def drain(tb_v, osem):
        pltpu.make_async_copy(
            tb_v, t_hbm.at[pl.ds(base_px, CH)], osem).wait()

    def body(m, carry):
        k0 = 2 * m

        @pl.when(m > 0)
        def _():
            drain(tbA, osemA)

        fill(k0, tbA)
        fire(k0, tbA, osemA)

        @pl.when(m > 0)
        def _():
            drain(tbB, osemB)

        fill(k0 + 1, tbB)
        fire(k0 + 1, tbB, osemB)
        return carry

    lax.fori_loop(0, NCHUNK // 2, body, 0)
    drain(tbA, osemA)
    drain(tbB, osemB)

    @pl.when(wid == NW - 1)
    def _():
        fz = jnp.full((16,), 0.0, _f32)

        def zgroup(rg, c2):
            pix = rg * 16 + iota
            for col in range(8):
                plsc.store_scatter(
                    tbA, [pix, jnp.full((16,), col, _i32)], fz)
            return c2

        lax.fori_loop(0, 512 // 16, zgroup, 0)
        pltpu.sync_copy(tbA.at[pl.ds(0, 512)], t_hbm.at[pl.ds(N, 512)])


def _sc_undistort_body(t_hbm, par_hbm, out_hbm,
                       par_v,
                       idxA, idxB, rowsTA, rowsBA, rowsTB, rowsBB,
                       w00A, w01A, w10A, w11A,
                       w00B, w01B, w10B, w11B,
                       o0A, o1A, o2A, o0B, o1B, o2B,
                       gsemA, gsemB, osemA, osemB):
    wid = lax.axis_index("s") * 2 + lax.axis_index("c")
    base_px = wid * PPW
    pltpu.sync_copy(par_hbm, par_v)
    iota = lax.iota(_i32, 16)

    bufs = (
        (idxA, (rowsTA, rowsBA), (w00A, w01A, w10A, w11A),
         (o0A, o1A, o2A), gsemA, osemA),
        (idxB, (rowsTB, rowsBB), (w00B, w01B, w10B, w11B),
         (o0B, o1B, o2B), gsemB, osemB),
    )

    iotaf4 = (iota * 4).astype(_f32)
    fone = jnp.full((16,), 1.0, _f32)
    fzero = jnp.full((16,), 0.0, _f32)

    def phase1(kk, par):
        idx_v, _, wv, _, _, _ = bufs[par]
        w00_v, w01_v, w10_v, w11_v = wv
        kq = par_v[0, :]
        e1x = par_v[2, :]
        e1y = par_v[3, :]
        ybase = (base_px + kk * CH) // W

        def pgroup(q, c2):
            yrow = (ybase + (q // 8)).astype(_f32)
            yup = yrow - e1y
            yup2 = yup * yup
            for j in range(4):
                s_base = q * 64 + j * 16
                x0 = ((q % 8) * 64 + j).astype(_f32)
                xup = (x0 + iotaf4) - e1x
                q2 = xup * xup + yup2
                r = 1.0 / (1.0 - kq * q2)
                xd = xup * r + e1x
                yd = yup * r + e1y
                valid = ((xd >= 0.0) & (xd <= float(W - 1))) \
                    & ((yd >= 0.0) & (yd <= float(H - 1)))
                xt = xd.astype(_i32)
                yt = yd.astype(_i32)
                ox = xd - xt.astype(_f32)
                oy = yd - yt.astype(_f32)
                vm = jnp.where(valid, fone, fzero)
                onyv = (1.0 - oy) * vm
                oyv = oy * vm
                onx = 1.0 - ox
                xf = jnp.minimum(jnp.maximum(xt, 0), W - 1)
                yf = jnp.minimum(jnp.maximum(yt, 0), H - 1)
                idx = lax.bitwise_or(lax.shift_left(yf, 9), xf)
                idx_v[pl.ds(s_base, 16)] = idx
                idx_v[pl.ds(CH + s_base, 16)] = idx + W
                w00_v[pl.ds(s_base, 16)] = onx * onyv
                w01_v[pl.ds(s_base, 16)] = ox * onyv
                w10_v[pl.ds(s_base, 16)] = onx * oyv
                w11_v[pl.ds(s_base, 16)] = ox * oyv
            return c2

        lax.fori_loop(0, NG, pgroup, 0)

    def fire_gather(par):
        idx_v, (rowsT, rowsB), _, _, gsem, _ = bufs[par]
        for half, rows_v in enumerate((rowsT, rowsB)):
            for i_ in range(NSUB):
                pltpu.async_copy(
                    t_hbm.at[idx_v.at[pl.ds(half * CH + i_ * SUBSZ, SUBSZ)]],
                    rows_v.at[pl.ds(i_ * SUBSZ, SUBSZ)], gsem)

    def drain_gather(par):
        idx_v, (rowsT, rowsB), _, _, gsem, _ = bufs[par]
        pltpu.make_async_copy(t_hbm.at[idx_v.at[pl.ds(0, CH)]],
                              rowsT, gsem).wait()
        pltpu.make_async_copy(t_hbm.at[idx_v.at[pl.ds(CH, CH)]],
                              rowsB, gsem).wait()

    def combine(kk, par):
        _, (rowsT, rowsB), wv, ov, _, osem = bufs[par]
        w00_v, w01_v, w10_v, w11_v = wv

        def cgroup(q, c2):
            words = [jnp.full((16,), 0, _i32) for _ in range(3)]
            for j in range(4):
                s_base = q * 64 + j * 16
                pix = s_base + iota
                w00 = w00_v[pl.ds(s_base, 16)]
                w01 = w01_v[pl.ds(s_base, 16)]
                w10 = w10_v[pl.ds(s_base, 16)]
                w11 = w11_v[pl.ds(s_base, 16)]
                for c_ in range(3):
                    v00 = plsc.load_gather(
                        rowsT, [pix, jnp.full((16,), c_, _i32)])
                    v01 = plsc.load_gather(
                        rowsT, [pix, jnp.full((16,), 4 + c_, _i32)])
                    v10 = plsc.load_gather(
                        rowsB, [pix, jnp.full((16,), c_, _i32)])
                    v11 = plsc.load_gather(
                        rowsB, [pix, jnp.full((16,), 4 + c_, _i32)])
                    acc = w00 * v00 + w01 * v01 + w10 * v10 + w11 * v11
                    byte = lax.bitwise_and(acc.astype(_i32), 255)
                    words[c_] = lax.bitwise_or(
                        words[c_], lax.shift_left(byte, 8 * j))
            ov[0][pl.ds(q * 64, 64)] = plsc.bitcast(words[0], jnp.uint8)
            ov[1][pl.ds(q * 64, 64)] = plsc.bitcast(words[1], jnp.uint8)
            ov[2][pl.ds(q * 64, 64)] = plsc.bitcast(words[2], jnp.uint8)
            return c2

        lax.fori_loop(0, NG, cgroup, 0)

    def fire_out(kk, par):
        _, _, _, ov, _, osem = bufs[par]
        wbase = wid * PPW + kk * CH
        for c_ in range(3):
            pltpu.async_copy(
                ov[c_], out_hbm.at[pl.ds(c_ * N + wbase, CH)],
                osem)

    def drain_out(par):
        _, _, _, ov, _, osem = bufs[par]
        for c_ in range(3):
            pltpu.make_async_copy(
                ov[c_], out_hbm.at[pl.ds(c_ * N, CH)],
                osem).wait()

    HALF = NCHUNK // 2

    phase1(0, 0)
    fire_gather(0)

    def body(m, carry):
        k0 = 2 * m
        k1 = k0 + 1

        phase1(k1, 1)
        fire_gather(1)

        drain_gather(0)

        @pl.when(m > 0)
        def _():
            drain_out(0)

        combine(k0, 0)
        fire_out(k0, 0)

        @pl.when(m < HALF - 1)
        def _():
            phase1(k0 + 2, 0)
            fire_gather(0)

        drain_gather(1)

        @pl.when(m > 0)
        def _():
            drain_out(1)

        combine(k1, 1)
        fire_out(k1, 1)
        return carry

    lax.fori_loop(0, HALF, body, 0)
    drain_out(0)
    drain_out(1)


def _sc_build(imp):
    f = pl.kernel(
        _sc_build_body,
        out_type=jax.ShapeDtypeStruct((N + W, 8), _f32),
        scratch_types=[
            pltpu.VMEM((CHBUF,), _f32),
            pltpu.VMEM((CHBUF,), _f32),
            pltpu.VMEM((CHBUF,), _f32),
            pltpu.VMEM((CH, 8), _f32),
            pltpu.VMEM((CH, 8), _f32),
            pltpu.SemaphoreType.DMA,
            pltpu.SemaphoreType.DMA,
        ],
        **_SC_PARAMS,
    )
    return f(imp)


@functools.partial(jax.jit, static_argnames=())
def _undistort_sc(t, params):
    f = pl.kernel(
        _sc_undistort_body,
        out_type=jax.ShapeDtypeStruct((C * N,), jnp.uint8),
        scratch_types=(
            [pltpu.VMEM((4, 16), _f32)]
            + [pltpu.VMEM((2 * CH,), _i32)] * 2
            + [pltpu.VMEM((CH, 8), _f32)] * 4
            + [pltpu.VMEM((CH,), _f32)] * 8
            + [pltpu.VMEM((CH,), jnp.uint8)] * 6
            + [pltpu.SemaphoreType.DMA] * 4
        ),
        **_SC_PARAMS,
    )
    return f(t, params)


def kernel(im_d, k, dx, dy):
    imp = im_d.reshape(-1)
    t = _sc_build(imp)
    params = jnp.stack([
        jnp.broadcast_to(k.reshape(1) * (1.0 / (W * W)), (16,)),
        jnp.zeros((16,), _f32),
        jnp.broadcast_to(dx.reshape(1) + (W / 2.0), (16,)),
        jnp.broadcast_to(dy.reshape(1) + (H / 2.0), (16,)),
    ]).astype(_f32)
    out = _undistort_sc(t, params)
    return out.reshape(C, H, W)
